# Initial kernel scaffold; baseline (speedup 1.0000x reference)
import jax
import jax.numpy as jnp
from jax.experimental import pallas as pl


def kernel(x_user, x_item, edge_follows, edge_rated_by, edge_rates, W0_follows, W0_rated_by, W0_rates, W1_follows, W1_rated_by, W1_rates, W2_follows, W2_rated_by, W2_rates):
    raise NotImplementedError("write your pallas kernel here")


# speedup vs baseline: 1.4600x; 1.4600x over previous
import functools

import jax
import jax.numpy as jnp
from jax import lax
from jax.experimental import pallas as pl
from jax.experimental.pallas import tpu as pltpu
from jax.experimental.pallas import tpu_sc as plsc

N = 10000
E = 160000
H = 128
OUT = 16

NC = 2
NS = 16
NW = NC * NS
CHUNK = 128
E_PAD = 163840
PER_TILE = E_PAD // NW
N_CHUNKS = PER_TILE // CHUNK
N_ACC = N + 8
Z_WIN = 640
Z_STEP = 624


def _make_agg(n_rel: int, d: int):
    mesh = plsc.VectorSubcoreMesh(core_axis_name="c", subcore_axis_name="s")

    @functools.partial(
        pl.kernel,
        out_type=jax.ShapeDtypeStruct((NC, N, d), jnp.float32),
        mesh=mesh,
        scratch_types=[
            pltpu.VMEM((CHUNK,), jnp.int32),
            pltpu.VMEM((CHUNK,), jnp.int32),
            pltpu.VMEM((CHUNK, d), jnp.float32),
            pltpu.VMEM_SHARED((N_ACC, d), jnp.float32),
            pltpu.SemaphoreType.DMA(()),
        ],
    )
    def agg(*refs):
        ins = refs[: 3 * n_rel + 1]
        out = refs[3 * n_rel + 1]
        src_v, dst_v, rows_v, acc, sem = refs[3 * n_rel + 2:]
        zeros_hbm = ins[3 * n_rel]

        cid = lax.axis_index("c")
        sid = lax.axis_index("s")
        wid = cid * NS + sid
        z0 = sid * Z_STEP

        pltpu.sync_copy(zeros_hbm.at[pl.ds(z0, Z_WIN)],
                        acc.at[pl.ds(z0, Z_WIN)])
        plsc.subcore_barrier()

        base = wid * PER_TILE
        for r in range(n_rel):
            tbl, src, dst = ins[3 * r], ins[3 * r + 1], ins[3 * r + 2]

            def body(j, _, tbl=tbl, src=src, dst=dst):
                off = base + j * CHUNK
                pltpu.sync_copy(src.at[pl.ds(off, CHUNK)], src_v)
                pltpu.sync_copy(dst.at[pl.ds(off, CHUNK)], dst_v)
                pltpu.async_copy(tbl.at[src_v], rows_v, sem).wait()
                pltpu.sync_copy(rows_v, acc.at[dst_v], add=True)
                return _

            lax.fori_loop(0, N_CHUNKS, body, None)

        plsc.subcore_barrier()
        pltpu.sync_copy(acc.at[pl.ds(z0, Z_WIN)],
                        out.at[cid, pl.ds(z0, Z_WIN)])

    return agg


_agg2_h = _make_agg(2, H)
_agg1_h = _make_agg(1, H)


def _proj_first_body(xu, xi, wf, wrb, wr, mf, mrb, mr):
    hu = xu[...]
    hi = xi[...]
    mf[...] = jnp.dot(hu, wf[...], preferred_element_type=jnp.float32)
    mr[...] = jnp.dot(hu, wr[...], preferred_element_type=jnp.float32)
    mrb[...] = jnp.dot(hi, wrb[...], preferred_element_type=jnp.float32)


def _proj_mid_body(up, ip, wf, wrb, wr, mf, mrb, mr):
    hu = jnp.maximum(up[0] + up[1], 0.0)
    hi = jnp.maximum(ip[0] + ip[1], 0.0)
    mf[...] = jnp.dot(hu, wf[...], preferred_element_type=jnp.float32)
    mr[...] = jnp.dot(hu, wr[...], preferred_element_type=jnp.float32)
    mrb[...] = jnp.dot(hi, wrb[...], preferred_element_type=jnp.float32)


def _combine_body(up, ip, hu, hi):
    hu[...] = jnp.maximum(up[0] + up[1], 0.0)
    hi[...] = jnp.maximum(ip[0] + ip[1], 0.0)


def _final_proj_body(af, arb, ar, wf, wrb, wr, hu, hi):
    au = jnp.dot(af[0] + af[1], wf[...], preferred_element_type=jnp.float32)
    au += jnp.dot(arb[0] + arb[1], wrb[...], preferred_element_type=jnp.float32)
    hu[...] = jnp.maximum(au, 0.0)
    ai = jnp.dot(ar[0] + ar[1], wr[...], preferred_element_type=jnp.float32)
    hi[...] = jnp.maximum(ai, 0.0)


def _m3_shapes(d):
    return [jax.ShapeDtypeStruct((N, d), jnp.float32)] * 3


_proj_first = pl.pallas_call(_proj_first_body, out_shape=_m3_shapes(H))
_proj_mid_h = pl.pallas_call(_proj_mid_body, out_shape=_m3_shapes(H))
_combine = pl.pallas_call(
    _combine_body,
    out_shape=[jax.ShapeDtypeStruct((N, H), jnp.float32)] * 2,
)
_final_proj = pl.pallas_call(
    _final_proj_body,
    out_shape=[jax.ShapeDtypeStruct((N, OUT), jnp.float32)] * 2,
)


def _pad_edges(e):
    pad = E_PAD - E
    src = jnp.concatenate([e[0], jnp.zeros((pad,), jnp.int32)])
    dst = jnp.concatenate([e[1], jnp.full((pad,), N, jnp.int32)])
    return src, dst


def kernel(x_user, x_item, edge_follows, edge_rated_by, edge_rates,
           W0_follows, W0_rated_by, W0_rates,
           W1_follows, W1_rated_by, W1_rates,
           W2_follows, W2_rated_by, W2_rates):
    sf, df = _pad_edges(edge_follows)
    srb, drb = _pad_edges(edge_rated_by)
    sr, dr = _pad_edges(edge_rates)
    z_h = jnp.zeros((N, H), jnp.float32)

    m_f, m_rb, m_r = _proj_first(x_user, x_item, W0_follows, W0_rated_by,
                                 W0_rates)
    u_p = _agg2_h(m_f, sf, df, m_rb, srb, drb, z_h)
    i_p = _agg1_h(m_r, sr, dr, z_h)

    m_f, m_rb, m_r = _proj_mid_h(u_p, i_p, W1_follows, W1_rated_by, W1_rates)
    u_p = _agg2_h(m_f, sf, df, m_rb, srb, drb, z_h)
    i_p = _agg1_h(m_r, sr, dr, z_h)

    h2u, h2i = _combine(u_p, i_p)
    a_f = _agg1_h(h2u, sf, df, z_h)
    a_rb = _agg1_h(h2i, srb, drb, z_h)
    a_r = _agg1_h(h2u, sr, dr, z_h)
    return _final_proj(a_f, a_rb, a_r, W2_follows, W2_rated_by, W2_rates)
